# Initial kernel scaffold; baseline (speedup 1.0000x reference)
import jax
import jax.numpy as jnp
from jax.experimental import pallas as pl


def kernel(x, W1, W2, W3, W4, W5):
    raise NotImplementedError("write your pallas kernel here")


# speedup vs baseline: 3.8347x; 3.8347x over previous
import functools
import jax
import jax.numpy as jnp
from jax.experimental import pallas as pl

N = 8192
K = 20
BN_EPS = 1e-5
NEG = -jnp.inf


def _leaky(y):
    return jnp.where(y >= 0, y, 0.2 * y)


def _act(y):
    return _leaky(y / jnp.sqrt(1.0 + BN_EPS))


def _knn_body(xrows_ref, xall_ref, ncol_ref, nrow_ref, idx_ref):
    xr = xrows_ref[...]
    xa = xall_ref[...]
    inner = jax.lax.dot_general(xr, xa, (((1,), (1,)), ((), ())),
                                preferred_element_type=jnp.float32,
                                precision=jax.lax.Precision.DEFAULT)
    D = (2.0 * inner - ncol_ref[...]) - nrow_ref[...]
    TM = D.shape[0]
    iota = jax.lax.broadcasted_iota(jnp.int32, (TM, N), 1)
    cols = []
    for _ in range(K):
        m = jnp.max(D, axis=1, keepdims=True)
        sel = jnp.min(jnp.where(D == m, iota, N), axis=1, keepdims=True)
        cols.append(sel)
        D = jnp.where(iota == sel, NEG, D)
    idx_ref[...] = jnp.concatenate(cols, axis=1)


@functools.partial(jax.jit, static_argnames=("tm",))
def _knn_idx(xT, nrm, tm=256):
    C = xT.shape[1]
    grid = (N // tm,)
    return pl.pallas_call(
        _knn_body,
        grid=grid,
        in_specs=[
            pl.BlockSpec((tm, C), lambda i: (i, 0)),
            pl.BlockSpec((N, C), lambda i: (0, 0)),
            pl.BlockSpec((1, N), lambda i: (0, 0)),
            pl.BlockSpec((tm, 1), lambda i: (i, 0)),
        ],
        out_specs=pl.BlockSpec((tm, K), lambda i: (i, 0)),
        out_shape=jax.ShapeDtypeStruct((N, K), jnp.int32),
    )(xT, xT, nrm[None, :], nrm[:, None])


def _conv_body(feat_ref, w_ref, out_ref):
    f = feat_ref[...]
    w = w_ref[...]
    y = jax.lax.dot_general(f, w, (((1,), (1,)), ((), ())),
                            preferred_element_type=jnp.float32,
                            precision=jax.lax.Precision.DEFAULT)
    tm = y.shape[0] // K
    m = jnp.max(y.reshape(tm, K, y.shape[1]), axis=1)
    out_ref[...] = _act(m)


@functools.partial(jax.jit, static_argnames=("tm",))
def _conv_max(feat, W, tm=128):
    O = W.shape[0]
    C2 = W.shape[1]
    grid = (N // tm,)
    return pl.pallas_call(
        _conv_body,
        grid=grid,
        in_specs=[
            pl.BlockSpec((tm * K, C2), lambda i: (i, 0)),
            pl.BlockSpec((O, C2), lambda i: (0, 0)),
        ],
        out_specs=pl.BlockSpec((tm, O), lambda i: (i, 0)),
        out_shape=jax.ShapeDtypeStruct((N, O), jnp.float32),
    )(feat, W)


def _edge_layer(xT, W):
    C = xT.shape[1]
    pad = (-C) % 8
    xp = jnp.pad(xT, ((0, 0), (0, pad))) if pad else xT
    nrm = jnp.sum(xT * xT, axis=1)
    idx = _knn_idx(xp, nrm)
    nb = jnp.take(xT, idx.reshape(-1), axis=0)
    center = jnp.repeat(xT, K, axis=0)
    feat = jnp.concatenate([nb - center, center], axis=1)
    return _conv_max(feat, W)


def kernel(x, W1, W2, W3, W4, W5):
    xT = x[0].T
    x1 = _edge_layer(xT, W1)
    x2 = _edge_layer(x1, W2)
    x3 = _edge_layer(x2, W3)
    x4 = _edge_layer(x3, W4)
    cat = jnp.concatenate([x1, x2, x3, x4], axis=1)
    x5 = _act(cat @ W5.T)
    xg = jnp.max(x5, axis=0)
    x5t = x5.T
    out = jnp.concatenate(
        [x5t, jnp.broadcast_to(xg[:, None], x5t.shape)], axis=0)
    return out[None]
